# baseline (speedup 1.0000x reference)
import jax
import jax.numpy as jnp
from jax import lax
from jax.experimental import pallas as pl
from jax.experimental.pallas import tpu as pltpu
from jax.experimental.pallas import tpu_sc as plsc

N_NODES = 2625
N_NODESP = 2632
USERS = 943
ITEMS = N_NODES - USERS
N_REL = 5
H0 = 500
H0P = 512
H1 = 75
H1P = 128
E = 200000

NC, NS = 2, 16
NW = NC * NS
EPAD = 200192
CHUNK = 64

AROWS = 2816
RPS = AROWS // NW
SENT = 4096

P = USERS * ITEMS
PTAIL = 22
PMAIN = P - PTAIL
PW_C = 49568
S31 = PMAIN - PW_C
CP = 4096
CP_TAIL = PW_C - 12 * CP

_GDN = lax.GatherDimensionNumbers(
    offset_dims=(), collapsed_slice_dims=(0,), start_index_map=(0,))


def _vgather(v, idx16):
    return lax.gather(v, idx16[:, None], _GDN, slice_sizes=(1,),
                      mode=lax.GatherScatterMode.PROMISE_IN_BOUNDS)


def _cumsum_body(x_ref, o_ref):
    w = x_ref[0]
    o_ref[0] = w
    for r in range(1, N_REL):
        w = w + x_ref[r]
        o_ref[r] = w


def _weight_table(ord_pad):
    return pl.pallas_call(
        _cumsum_body,
        grid=(7,),
        in_specs=[pl.BlockSpec((N_REL, 376, H0P), lambda i: (0, i, 0))],
        out_specs=pl.BlockSpec((N_REL, 376, H0P), lambda i: (0, i, 0)),
        out_shape=jax.ShapeDtypeStruct((N_REL, N_NODESP, H0P), jnp.float32),
    )(ord_pad)


_sc_mesh = None


def _get_mesh():
    global _sc_mesh
    if _sc_mesh is None:
        _sc_mesh = plsc.VectorSubcoreMesh(
            core_axis_name="c", subcore_axis_name="s",
            num_cores=NC, num_subcores=NS)
    return _sc_mesh


def _rgc_body(w_hbm, gidx_hbm, dst_hbm, norm_hbm, bnds_hbm, aggr_hbm,
              idx_v, dst_v, norm_v, nmk_v, dl_v, rows_v, acc, bv, sem):
    c = lax.axis_index("c")
    s = lax.axis_index("s")
    wid = c * NS + s
    lo = pl.multiple_of(wid * RPS, RPS)
    zero16 = jnp.zeros((16,), jnp.float32)

    def zrow(i, _):
        for k in range(H0P // 16):
            acc[i, pl.ds(k * 16, 16)] = zero16
        return 0
    lax.fori_loop(0, RPS, zrow, 0)

    pltpu.sync_copy(bnds_hbm.at[pl.ds(pl.multiple_of(wid * 8, 8), 16)], bv)
    bb = bv[pl.ds(0, 16)]
    start = bb[0]
    end = bb[1]
    astart = pl.multiple_of((start >> 6) << 6, CHUNK)
    nchunks = (end - astart + 63) >> 6

    def chunk(g, _):
        off = pl.multiple_of(astart + g * CHUNK, CHUNK)
        pltpu.sync_copy(gidx_hbm.at[pl.ds(off, CHUNK)], idx_v)
        pltpu.sync_copy(dst_hbm.at[pl.ds(off, CHUNK)], dst_v)
        pltpu.sync_copy(norm_hbm.at[pl.ds(off, CHUNK)], norm_v)
        pltpu.async_copy(w_hbm.at[idx_v], rows_v, sem).wait()
        for q in range(CHUNK // 16):
            dvec = dst_v[pl.ds(q * 16, 16)]
            nvec = norm_v[pl.ds(q * 16, 16)]
            inm = (dvec >= lo) & (dvec < lo + RPS)
            nmk_v[pl.ds(q * 16, 16)] = jnp.where(inm, nvec, 0.0)
            dl_v[pl.ds(q * 16, 16)] = jnp.clip(dvec - lo, 0, RPS - 1)

        def egroup(g4, _):
            dvec = dl_v[pl.ds(g4 * 16, 16)]
            nvec = nmk_v[pl.ds(g4 * 16, 16)]
            for l in range(16):
                d = dvec[l]
                nv = _vgather(nvec, jnp.full((16,), l, jnp.int32))
                el = g4 * 16 + l

                def kcol(k, _):
                    for u in range(4):
                        off2 = k * 64 + u * 16
                        v = acc[d, pl.ds(off2, 16)]
                        rv = rows_v[el, pl.ds(off2, 16)]
                        acc[d, pl.ds(off2, 16)] = v + rv * nv
                    return 0
                lax.fori_loop(0, H0P // 64, kcol, 0)
            return 0
        lax.fori_loop(0, CHUNK // 16, egroup, 0)
        return 0
    lax.fori_loop(0, nchunks, chunk, 0)

    pltpu.sync_copy(acc, aggr_hbm.at[pl.ds(lo, RPS)])


def _rgc(wtable, gidx_s, dst_s, norm_s, bnds):
    return pl.kernel(
        _rgc_body,
        out_type=jax.ShapeDtypeStruct((AROWS, H0P), jnp.float32),
        mesh=_get_mesh(),
        scratch_types=[
            pltpu.VMEM((CHUNK,), jnp.int32),
            pltpu.VMEM((CHUNK,), jnp.int32),
            pltpu.VMEM((CHUNK,), jnp.float32),
            pltpu.VMEM((CHUNK,), jnp.float32),
            pltpu.VMEM((CHUNK,), jnp.int32),
            pltpu.VMEM((CHUNK, H0P), jnp.float32),
            pltpu.VMEM((RPS, H0P), jnp.float32),
            pltpu.VMEM((16,), jnp.int32),
            pltpu.SemaphoreType.DMA,
        ],
    )(wtable, gidx_s, dst_s, norm_s, bnds)


def _dec_body(coefs_sm, aggr_ref, w_ref, basis_ref, t_ref, feat_sc):
    r = pl.program_id(0)

    @pl.when(r == 0)
    def _():
        f = jnp.maximum(aggr_ref[0:2632, :], 0.0)
        f75 = jnp.dot(f, w_ref[...], preferred_element_type=jnp.float32,
                      precision=lax.Precision.HIGHEST)
        feat_sc[...] = jnp.maximum(f75, 0.0)

    c0 = coefs_sm[r, 0]
    c1 = coefs_sm[r, 1]
    q = c0 * basis_ref[0] + c1 * basis_ref[1]
    u = feat_sc[0:944, :]
    it = feat_sc[944:2632, :]
    zu = jnp.dot(u, q, preferred_element_type=jnp.float32,
                 precision=lax.Precision.HIGHEST)
    t = lax.dot_general(zu, it, (((1,), (1,)), ((), ())),
                        preferred_element_type=jnp.float32,
                        precision=lax.Precision.HIGHEST)
    t_ref[0] = t[0:USERS, 0:ITEMS]


def _decode(coefs, aggr, w_pad, basis_pad):
    return pl.pallas_call(
        _dec_body,
        grid=(N_REL,),
        in_specs=[
            pl.BlockSpec(memory_space=pltpu.SMEM),
            pl.BlockSpec((AROWS, H0P), lambda r: (0, 0)),
            pl.BlockSpec((H0P, H1P), lambda r: (0, 0)),
            pl.BlockSpec((2, H1P, H1P), lambda r: (0, 0, 0)),
        ],
        out_specs=pl.BlockSpec((1, USERS, ITEMS), lambda r: (r, 0, 0)),
        out_shape=jax.ShapeDtypeStruct((N_REL, USERS, ITEMS), jnp.float32),
        scratch_shapes=[pltpu.VMEM((2632, H1P), jnp.float32)],
    )(coefs, aggr, w_pad, basis_pad)


_SH = [(6 * r) % 8 for r in range(N_REL)]


def _divmod5(x16):
    q = (x16 * 52429) >> 18
    return q, x16 - q * N_REL


def _il_body(t_hbm, o_hbm, pv0, pv1, pv2, pv3, pv4, ov):
    pv = [pv0, pv1, pv2, pv3, pv4]
    c = lax.axis_index("c")
    s = lax.axis_index("s")
    wid = c * NS + s
    sw = jnp.minimum(wid * PW_C, S31)
    ids = lax.iota(jnp.int32, 16)
    pi_c, pr_c = [], []
    for m in range(N_REL):
        q, rr = _divmod5(ids + 16 * m)
        pi_c.append(q)
        pr_c.append(rr)

    def do_chunk(off, cp):
        for r in range(N_REL):
            a0 = r * P + off - _SH[r]
            pltpu.sync_copy(t_hbm.at[pl.ds(a0, cp + 16)],
                            pv[r].at[pl.ds(0, cp + 16)])

        def jloop(j, _):
            vs = [pv[r][pl.ds(j * 16 + _SH[r], 16)] for r in range(N_REL)]
            for m in range(N_REL):
                o = jnp.zeros((16,), jnp.float32)
                for r in range(N_REL):
                    g = _vgather(vs[r], pi_c[m])
                    o = jnp.where(pr_c[m] == r, g, o)
                ov[pl.ds(j * 80 + m * 16, 16)] = o
            return 0
        lax.fori_loop(0, cp // 16, jloop, 0)
        pltpu.sync_copy(ov.at[pl.ds(0, cp * N_REL)],
                        o_hbm.at[pl.ds(off * N_REL, cp * N_REL)])

    def cloop(ci, _):
        do_chunk(sw + ci * CP, CP)
        return 0
    lax.fori_loop(0, 12, cloop, 0)
    do_chunk(sw + 12 * CP, CP_TAIL)

    @pl.when(wid == 0)
    def _():
        for r in range(N_REL):
            a0 = r * P + PMAIN - _SH[r]
            pltpu.sync_copy(t_hbm.at[pl.ds(a0, PTAIL + _SH[r])],
                            pv[r].at[pl.ds(0, PTAIL + _SH[r])])
        for j in range(2):
            vs = [pv[r][pl.ds(j * 16 + _SH[r], 16)] for r in range(N_REL)]
            for m in range(N_REL):
                mm = j * N_REL + m
                if mm > 6:
                    continue
                pi, pr = _divmod5(ids + 16 * mm)
                pi = jnp.maximum(pi - 16 * j, 0)
                o = jnp.zeros((16,), jnp.float32)
                for r in range(N_REL):
                    g = _vgather(vs[r], pi)
                    o = jnp.where(pr == r, g, o)
                ov[pl.ds(mm * 16, 16)] = o
        pltpu.sync_copy(ov.at[pl.ds(0, PTAIL * N_REL)],
                        o_hbm.at[pl.ds(PMAIN * N_REL, PTAIL * N_REL)])


def _interleave(t_flat):
    return pl.kernel(
        _il_body,
        out_type=jax.ShapeDtypeStruct((P * N_REL,), jnp.float32),
        mesh=_get_mesh(),
        scratch_types=[
            pltpu.VMEM((CP + 16,), jnp.float32),
            pltpu.VMEM((CP + 16,), jnp.float32),
            pltpu.VMEM((CP + 16,), jnp.float32),
            pltpu.VMEM((CP + 16,), jnp.float32),
            pltpu.VMEM((CP + 16,), jnp.float32),
            pltpu.VMEM((CP * N_REL,), jnp.float32),
        ],
    )(t_flat)


def kernel(x, edge_index, edge_type, edge_norm, ord_basis, dense_w,
           basis_matrix, coefs):
    src = jnp.take(x, edge_index[0], axis=0)
    gidx = edge_type * N_NODESP + src
    dst = edge_index[1]
    dst2 = dst + (dst >= USERS).astype(jnp.int32)
    gidx_p = jnp.pad(gidx, (0, EPAD - E))
    dst_p = jnp.pad(dst2, (0, EPAD - E), constant_values=SENT)
    norm_p = jnp.pad(edge_norm, (0, EPAD - E))
    order = jnp.argsort(dst_p)
    gidx_s = jnp.take(gidx_p, order)
    dst_s = jnp.take(dst_p, order)
    norm_s = jnp.take(norm_p, order)
    b33 = jnp.searchsorted(dst_s, jnp.arange(0, (NW + 1) * RPS, RPS,
                                             dtype=jnp.int32)).astype(jnp.int32)
    bnds = jnp.pad(jnp.stack([b33[:NW], b33[1:]], axis=1),
                   ((0, 1), (0, 6))).reshape(-1)

    ord_pad = jnp.pad(ord_basis.reshape(N_REL, N_NODES, H0),
                      ((0, 0), (0, N_NODESP - N_NODES), (0, H0P - H0)))
    w_pad = jnp.pad(dense_w, ((0, H0P - H0), (0, H1P - H1)))
    basis_pad = jnp.pad(basis_matrix.reshape(2, H1, H1),
                        ((0, 0), (0, H1P - H1), (0, H1P - H1)))

    wtable = _weight_table(ord_pad).reshape(N_REL * N_NODESP, H0P)
    aggr = _rgc(wtable, gidx_s, dst_s, norm_s, bnds)
    t = _decode(coefs, aggr, w_pad, basis_pad)
    out = _interleave(t.reshape(N_REL * P))
    return out.reshape(P, N_REL)


# speedup vs baseline: 1.4076x; 1.4076x over previous
import jax
import jax.numpy as jnp
from jax import lax
from jax.experimental import pallas as pl
from jax.experimental.pallas import tpu as pltpu
from jax.experimental.pallas import tpu_sc as plsc

N_NODES = 2625
N_NODESP = 2632
USERS = 943
ITEMS = N_NODES - USERS
N_REL = 5
H0 = 500
H0P = 512
H1 = 75
H1P = 128
E = 200000

NC, NS = 2, 16
NW = NC * NS
EPAD = 200192
CHUNK = 64

AROWS = 2816
RPS = AROWS // NW
SENT = 4096

P = USERS * ITEMS
PTAIL = 22
PMAIN = P - PTAIL
PW_C = 49568
S31 = PMAIN - PW_C
CP = 4096
CP_TAIL = PW_C - 12 * CP

_GDN = lax.GatherDimensionNumbers(
    offset_dims=(), collapsed_slice_dims=(0,), start_index_map=(0,))


def _vgather(v, idx16):
    return lax.gather(v, idx16[:, None], _GDN, slice_sizes=(1,),
                      mode=lax.GatherScatterMode.PROMISE_IN_BOUNDS)


def _cumsum_body(x_ref, o_ref):
    w = x_ref[0]
    o_ref[0] = w
    for r in range(1, N_REL):
        w = w + x_ref[r]
        o_ref[r] = w


def _weight_table(ord_pad):
    return pl.pallas_call(
        _cumsum_body,
        grid=(7,),
        in_specs=[pl.BlockSpec((N_REL, 376, H0P), lambda i: (0, i, 0))],
        out_specs=pl.BlockSpec((N_REL, 376, H0P), lambda i: (0, i, 0)),
        out_shape=jax.ShapeDtypeStruct((N_REL, N_NODESP, H0P), jnp.float32),
    )(ord_pad)


_sc_mesh = None


def _get_mesh():
    global _sc_mesh
    if _sc_mesh is None:
        _sc_mesh = plsc.VectorSubcoreMesh(
            core_axis_name="c", subcore_axis_name="s",
            num_cores=NC, num_subcores=NS)
    return _sc_mesh


def _rgc_body(w_hbm, gidx_hbm, dst_hbm, norm_hbm, bnds_hbm, aggr_hbm,
              idx_v, dst_v, norm_v, nmk_v, dl_v, rows_v, acc, bv, sem):
    c = lax.axis_index("c")
    s = lax.axis_index("s")
    wid = c * NS + s
    lo = pl.multiple_of(wid * RPS, RPS)
    zero16 = jnp.zeros((16,), jnp.float32)

    def zrow(i, _):
        for k in range(H0P // 16):
            acc[i, pl.ds(k * 16, 16)] = zero16
        return 0
    lax.fori_loop(0, RPS, zrow, 0)

    pltpu.sync_copy(bnds_hbm.at[pl.ds(pl.multiple_of(wid * 8, 8), 16)], bv)
    bb = bv[pl.ds(0, 16)]
    start = bb[0]
    end = bb[1]
    astart = pl.multiple_of((start >> 6) << 6, CHUNK)
    nchunks = (end - astart + 63) >> 6

    def chunk(g, _):
        off = pl.multiple_of(astart + g * CHUNK, CHUNK)
        pltpu.sync_copy(gidx_hbm.at[pl.ds(off, CHUNK)], idx_v)
        pltpu.sync_copy(dst_hbm.at[pl.ds(off, CHUNK)], dst_v)
        pltpu.sync_copy(norm_hbm.at[pl.ds(off, CHUNK)], norm_v)
        pltpu.async_copy(w_hbm.at[idx_v], rows_v, sem).wait()
        for q in range(CHUNK // 16):
            dvec = dst_v[pl.ds(q * 16, 16)]
            nvec = norm_v[pl.ds(q * 16, 16)]
            inm = (dvec >= lo) & (dvec < lo + RPS)
            nmk_v[pl.ds(q * 16, 16)] = jnp.where(inm, nvec, 0.0)
            dl_v[pl.ds(q * 16, 16)] = jnp.clip(dvec - lo, 0, RPS - 1)

        def egroup(g4, _):
            dvec = dl_v[pl.ds(g4 * 16, 16)]
            nvec = nmk_v[pl.ds(g4 * 16, 16)]
            for l in range(16):
                d = dvec[l]
                nv = _vgather(nvec, jnp.full((16,), l, jnp.int32))
                el = g4 * 16 + l

                @plsc.parallel_loop(0, H0P // 16, unroll=4)
                def _(k):
                    off2 = k * 16
                    v = acc[d, pl.ds(off2, 16)]
                    rv = rows_v[el, pl.ds(off2, 16)]
                    acc[d, pl.ds(off2, 16)] = v + rv * nv
            return 0
        lax.fori_loop(0, CHUNK // 16, egroup, 0)
        return 0
    lax.fori_loop(0, nchunks, chunk, 0)

    pltpu.sync_copy(acc, aggr_hbm.at[pl.ds(lo, RPS)])


def _rgc(wtable, gidx_s, dst_s, norm_s, bnds):
    return pl.kernel(
        _rgc_body,
        out_type=jax.ShapeDtypeStruct((AROWS, H0P), jnp.float32),
        mesh=_get_mesh(),
        scratch_types=[
            pltpu.VMEM((CHUNK,), jnp.int32),
            pltpu.VMEM((CHUNK,), jnp.int32),
            pltpu.VMEM((CHUNK,), jnp.float32),
            pltpu.VMEM((CHUNK,), jnp.float32),
            pltpu.VMEM((CHUNK,), jnp.int32),
            pltpu.VMEM((CHUNK, H0P), jnp.float32),
            pltpu.VMEM((RPS, H0P), jnp.float32),
            pltpu.VMEM((16,), jnp.int32),
            pltpu.SemaphoreType.DMA,
        ],
    )(wtable, gidx_s, dst_s, norm_s, bnds)


def _dec_body(coefs_sm, aggr_ref, w_ref, basis_ref, t_ref, feat_sc):
    r = pl.program_id(0)

    @pl.when(r == 0)
    def _():
        f = jnp.maximum(aggr_ref[0:2632, :], 0.0)
        f75 = jnp.dot(f, w_ref[...], preferred_element_type=jnp.float32,
                      precision=lax.Precision.HIGHEST)
        feat_sc[...] = jnp.maximum(f75, 0.0)

    c0 = coefs_sm[r, 0]
    c1 = coefs_sm[r, 1]
    q = c0 * basis_ref[0] + c1 * basis_ref[1]
    u = feat_sc[0:944, :]
    it = feat_sc[944:2632, :]
    zu = jnp.dot(u, q, preferred_element_type=jnp.float32,
                 precision=lax.Precision.HIGHEST)
    t = lax.dot_general(zu, it, (((1,), (1,)), ((), ())),
                        preferred_element_type=jnp.float32,
                        precision=lax.Precision.HIGHEST)
    t_ref[0] = t[0:USERS, 0:ITEMS]


def _decode(coefs, aggr, w_pad, basis_pad):
    return pl.pallas_call(
        _dec_body,
        grid=(N_REL,),
        in_specs=[
            pl.BlockSpec(memory_space=pltpu.SMEM),
            pl.BlockSpec((AROWS, H0P), lambda r: (0, 0)),
            pl.BlockSpec((H0P, H1P), lambda r: (0, 0)),
            pl.BlockSpec((2, H1P, H1P), lambda r: (0, 0, 0)),
        ],
        out_specs=pl.BlockSpec((1, USERS, ITEMS), lambda r: (r, 0, 0)),
        out_shape=jax.ShapeDtypeStruct((N_REL, USERS, ITEMS), jnp.float32),
        scratch_shapes=[pltpu.VMEM((2632, H1P), jnp.float32)],
    )(coefs, aggr, w_pad, basis_pad)


_SH = [(6 * r) % 8 for r in range(N_REL)]


def _divmod5(x16):
    q = (x16 * 52429) >> 18
    return q, x16 - q * N_REL


def _il_body(t_hbm, o_hbm, pv0, pv1, pv2, pv3, pv4, ov):
    pv = [pv0, pv1, pv2, pv3, pv4]
    c = lax.axis_index("c")
    s = lax.axis_index("s")
    wid = c * NS + s
    sw = jnp.minimum(wid * PW_C, S31)
    ids = lax.iota(jnp.int32, 16)
    pi_c, pr_c = [], []
    for m in range(N_REL):
        q, rr = _divmod5(ids + 16 * m)
        pi_c.append(q)
        pr_c.append(rr)

    def do_chunk(off, cp):
        for r in range(N_REL):
            a0 = r * P + off - _SH[r]
            pltpu.sync_copy(t_hbm.at[pl.ds(a0, cp + 16)],
                            pv[r].at[pl.ds(0, cp + 16)])

        def jloop(j, _):
            vs = [pv[r][pl.ds(j * 16 + _SH[r], 16)] for r in range(N_REL)]
            for m in range(N_REL):
                o = jnp.zeros((16,), jnp.float32)
                for r in range(N_REL):
                    g = _vgather(vs[r], pi_c[m])
                    o = jnp.where(pr_c[m] == r, g, o)
                ov[pl.ds(j * 80 + m * 16, 16)] = o
            return 0
        lax.fori_loop(0, cp // 16, jloop, 0)
        pltpu.sync_copy(ov.at[pl.ds(0, cp * N_REL)],
                        o_hbm.at[pl.ds(off * N_REL, cp * N_REL)])

    def cloop(ci, _):
        do_chunk(sw + ci * CP, CP)
        return 0
    lax.fori_loop(0, 12, cloop, 0)
    do_chunk(sw + 12 * CP, CP_TAIL)

    @pl.when(wid == 0)
    def _():
        for r in range(N_REL):
            a0 = r * P + PMAIN - _SH[r]
            pltpu.sync_copy(t_hbm.at[pl.ds(a0, PTAIL + _SH[r])],
                            pv[r].at[pl.ds(0, PTAIL + _SH[r])])
        for j in range(2):
            vs = [pv[r][pl.ds(j * 16 + _SH[r], 16)] for r in range(N_REL)]
            for m in range(N_REL):
                mm = j * N_REL + m
                if mm > 6:
                    continue
                pi, pr = _divmod5(ids + 16 * mm)
                pi = jnp.maximum(pi - 16 * j, 0)
                o = jnp.zeros((16,), jnp.float32)
                for r in range(N_REL):
                    g = _vgather(vs[r], pi)
                    o = jnp.where(pr == r, g, o)
                ov[pl.ds(mm * 16, 16)] = o
        pltpu.sync_copy(ov.at[pl.ds(0, PTAIL * N_REL)],
                        o_hbm.at[pl.ds(PMAIN * N_REL, PTAIL * N_REL)])


def _interleave(t_flat):
    return pl.kernel(
        _il_body,
        out_type=jax.ShapeDtypeStruct((P * N_REL,), jnp.float32),
        mesh=_get_mesh(),
        scratch_types=[
            pltpu.VMEM((CP + 16,), jnp.float32),
            pltpu.VMEM((CP + 16,), jnp.float32),
            pltpu.VMEM((CP + 16,), jnp.float32),
            pltpu.VMEM((CP + 16,), jnp.float32),
            pltpu.VMEM((CP + 16,), jnp.float32),
            pltpu.VMEM((CP * N_REL,), jnp.float32),
        ],
    )(t_flat)


def kernel(x, edge_index, edge_type, edge_norm, ord_basis, dense_w,
           basis_matrix, coefs):
    src = jnp.take(x, edge_index[0], axis=0)
    gidx = edge_type * N_NODESP + src
    dst = edge_index[1]
    dst2 = dst + (dst >= USERS).astype(jnp.int32)
    gidx_p = jnp.pad(gidx, (0, EPAD - E))
    dst_p = jnp.pad(dst2, (0, EPAD - E), constant_values=SENT)
    norm_p = jnp.pad(edge_norm, (0, EPAD - E))
    order = jnp.argsort(dst_p)
    gidx_s = jnp.take(gidx_p, order)
    dst_s = jnp.take(dst_p, order)
    norm_s = jnp.take(norm_p, order)
    b33 = jnp.searchsorted(dst_s, jnp.arange(0, (NW + 1) * RPS, RPS,
                                             dtype=jnp.int32)).astype(jnp.int32)
    bnds = jnp.pad(jnp.stack([b33[:NW], b33[1:]], axis=1),
                   ((0, 1), (0, 6))).reshape(-1)

    ord_pad = jnp.pad(ord_basis.reshape(N_REL, N_NODES, H0),
                      ((0, 0), (0, N_NODESP - N_NODES), (0, H0P - H0)))
    w_pad = jnp.pad(dense_w, ((0, H0P - H0), (0, H1P - H1)))
    basis_pad = jnp.pad(basis_matrix.reshape(2, H1, H1),
                        ((0, 0), (0, H1P - H1), (0, H1P - H1)))

    wtable = _weight_table(ord_pad).reshape(N_REL * N_NODESP, H0P)
    aggr = _rgc(wtable, gidx_s, dst_s, norm_s, bnds)
    t = _decode(coefs, aggr, w_pad, basis_pad)
    out = _interleave(t.reshape(N_REL * P))
    return out.reshape(P, N_REL)


# speedup vs baseline: 1.4209x; 1.0094x over previous
import jax
import jax.numpy as jnp
from jax import lax
from jax.experimental import pallas as pl
from jax.experimental.pallas import tpu as pltpu
from jax.experimental.pallas import tpu_sc as plsc

N_NODES = 2625
N_NODESP = 2632
USERS = 943
ITEMS = N_NODES - USERS
N_REL = 5
H0 = 500
H0P = 512
H1 = 75
H1P = 128
E = 200000

NC, NS = 2, 16
NW = NC * NS
EPAD = 200192
CHUNK = 64

AROWS = 2816
RPS = AROWS // NW
SENT = 4096

P = USERS * ITEMS
PTAIL = 22
PMAIN = P - PTAIL
PW_C = 49568
S31 = PMAIN - PW_C
CP = 4096
CP_TAIL = PW_C - 12 * CP

_GDN = lax.GatherDimensionNumbers(
    offset_dims=(), collapsed_slice_dims=(0,), start_index_map=(0,))


def _vgather(v, idx16):
    return lax.gather(v, idx16[:, None], _GDN, slice_sizes=(1,),
                      mode=lax.GatherScatterMode.PROMISE_IN_BOUNDS)


def _cumsum_body(x_ref, o_ref):
    w = x_ref[0]
    o_ref[0] = w
    for r in range(1, N_REL):
        w = w + x_ref[r]
        o_ref[r] = w


def _weight_table(ord_pad):
    return pl.pallas_call(
        _cumsum_body,
        grid=(7,),
        in_specs=[pl.BlockSpec((N_REL, 376, H0P), lambda i: (0, i, 0))],
        out_specs=pl.BlockSpec((N_REL, 376, H0P), lambda i: (0, i, 0)),
        out_shape=jax.ShapeDtypeStruct((N_REL, N_NODESP, H0P), jnp.float32),
    )(ord_pad)


_sc_mesh = None


def _get_mesh():
    global _sc_mesh
    if _sc_mesh is None:
        _sc_mesh = plsc.VectorSubcoreMesh(
            core_axis_name="c", subcore_axis_name="s",
            num_cores=NC, num_subcores=NS)
    return _sc_mesh


def _rgc_body(w_hbm, gidx_hbm, dst_hbm, norm_hbm, bnds_hbm, aggr_hbm,
              idx_v, dst_v, norm_v, nmk_v, dl_v, rows_v, acc, bv, sem):
    c = lax.axis_index("c")
    s = lax.axis_index("s")
    wid = c * NS + s
    lo = pl.multiple_of(wid * RPS, RPS)
    zero16 = jnp.zeros((16,), jnp.float32)

    def zrow(i, _):
        for k in range(H0P // 16):
            acc[i, pl.ds(k * 16, 16)] = zero16
        return 0
    lax.fori_loop(0, RPS, zrow, 0)

    pltpu.sync_copy(bnds_hbm.at[pl.ds(pl.multiple_of(wid * 8, 8), 16)], bv)
    bb = bv[pl.ds(0, 16)]
    start = bb[0]
    end = bb[1]
    astart = pl.multiple_of((start >> 6) << 6, CHUNK)
    nchunks = (end - astart + 63) >> 6

    def chunk(g, _):
        off = pl.multiple_of(astart + g * CHUNK, CHUNK)
        pltpu.sync_copy(gidx_hbm.at[pl.ds(off, CHUNK)], idx_v)
        pltpu.sync_copy(dst_hbm.at[pl.ds(off, CHUNK)], dst_v)
        pltpu.sync_copy(norm_hbm.at[pl.ds(off, CHUNK)], norm_v)
        pltpu.async_copy(w_hbm.at[idx_v], rows_v, sem).wait()
        for q in range(CHUNK // 16):
            dvec = dst_v[pl.ds(q * 16, 16)]
            nvec = norm_v[pl.ds(q * 16, 16)]
            inm = (dvec >= lo) & (dvec < lo + RPS)
            nmk_v[pl.ds(q * 16, 16)] = jnp.where(inm, nvec, 0.0)
            dl_v[pl.ds(q * 16, 16)] = jnp.clip(dvec - lo, 0, RPS - 1)

        def egroup(g4, _):
            dvec = dl_v[pl.ds(g4 * 16, 16)]
            nvec = nmk_v[pl.ds(g4 * 16, 16)]
            for l in range(16):
                d = dvec[l]
                nv = _vgather(nvec, jnp.full((16,), l, jnp.int32))
                el = g4 * 16 + l

                @plsc.parallel_loop(0, H0P // 16, unroll=4)
                def _(k):
                    off2 = k * 16
                    v = acc[d, pl.ds(off2, 16)]
                    rv = rows_v[el, pl.ds(off2, 16)]
                    acc[d, pl.ds(off2, 16)] = v + rv * nv
            return 0
        lax.fori_loop(0, CHUNK // 16, egroup, 0)
        return 0
    lax.fori_loop(0, nchunks, chunk, 0)

    pltpu.sync_copy(acc, aggr_hbm.at[pl.ds(lo, RPS)])


def _rgc(wtable, gidx_s, dst_s, norm_s, bnds):
    return pl.kernel(
        _rgc_body,
        out_type=jax.ShapeDtypeStruct((AROWS, H0P), jnp.float32),
        mesh=_get_mesh(),
        scratch_types=[
            pltpu.VMEM((CHUNK,), jnp.int32),
            pltpu.VMEM((CHUNK,), jnp.int32),
            pltpu.VMEM((CHUNK,), jnp.float32),
            pltpu.VMEM((CHUNK,), jnp.float32),
            pltpu.VMEM((CHUNK,), jnp.int32),
            pltpu.VMEM((CHUNK, H0P), jnp.float32),
            pltpu.VMEM((RPS, H0P), jnp.float32),
            pltpu.VMEM((16,), jnp.int32),
            pltpu.SemaphoreType.DMA,
        ],
    )(wtable, gidx_s, dst_s, norm_s, bnds)


def _dec_body(coefs_sm, aggr_ref, w_ref, basis_ref, t_ref, feat_sc):
    r = pl.program_id(0)

    @pl.when(r == 0)
    def _():
        f = jnp.maximum(aggr_ref[0:2632, :], 0.0)
        f75 = jnp.dot(f, w_ref[...], preferred_element_type=jnp.float32,
                      precision=lax.Precision.HIGHEST)
        feat_sc[...] = jnp.maximum(f75, 0.0)

    c0 = coefs_sm[r, 0]
    c1 = coefs_sm[r, 1]
    q = c0 * basis_ref[0] + c1 * basis_ref[1]
    u = feat_sc[0:944, :]
    it = feat_sc[944:2632, :]
    zu = jnp.dot(u, q, preferred_element_type=jnp.float32,
                 precision=lax.Precision.HIGHEST)
    t = lax.dot_general(zu, it, (((1,), (1,)), ((), ())),
                        preferred_element_type=jnp.float32,
                        precision=lax.Precision.HIGHEST)
    t_ref[0] = t[0:USERS, 0:ITEMS]


def _decode(coefs, aggr, w_pad, basis_pad):
    return pl.pallas_call(
        _dec_body,
        grid=(N_REL,),
        in_specs=[
            pl.BlockSpec(memory_space=pltpu.SMEM),
            pl.BlockSpec((AROWS, H0P), lambda r: (0, 0)),
            pl.BlockSpec((H0P, H1P), lambda r: (0, 0)),
            pl.BlockSpec((2, H1P, H1P), lambda r: (0, 0, 0)),
        ],
        out_specs=pl.BlockSpec((1, USERS, ITEMS), lambda r: (r, 0, 0)),
        out_shape=jax.ShapeDtypeStruct((N_REL, USERS, ITEMS), jnp.float32),
        scratch_shapes=[pltpu.VMEM((2632, H1P), jnp.float32)],
    )(coefs, aggr, w_pad, basis_pad)


_SH = [(6 * r) % 8 for r in range(N_REL)]


def _divmod5(x16):
    q = (x16 * 52429) >> 18
    return q, x16 - q * N_REL


def _il_body(t_hbm, o_hbm, pv0, pv1, pv2, pv3, pv4, ov):
    pv = [pv0, pv1, pv2, pv3, pv4]
    c = lax.axis_index("c")
    s = lax.axis_index("s")
    wid = c * NS + s
    sw = jnp.minimum(wid * PW_C, S31)
    ids = lax.iota(jnp.int32, 16)
    pi_c, pr_c = [], []
    for m in range(N_REL):
        q, rr = _divmod5(ids + 16 * m)
        pi_c.append(q)
        pr_c.append(rr)

    def do_chunk(off, cp):
        for r in range(N_REL):
            a0 = r * P + off - _SH[r]
            pltpu.sync_copy(t_hbm.at[pl.ds(a0, cp + 16)],
                            pv[r].at[pl.ds(0, cp + 16)])

        def jloop(j, _):
            vs = [pv[r][pl.ds(j * 16 + _SH[r], 16)] for r in range(N_REL)]
            for m in range(N_REL):
                o = jnp.zeros((16,), jnp.float32)
                for r in range(N_REL):
                    g = _vgather(vs[r], pi_c[m])
                    o = jnp.where(pr_c[m] == r, g, o)
                ov[pl.ds(j * 80 + m * 16, 16)] = o
            return 0
        lax.fori_loop(0, cp // 16, jloop, 0)
        pltpu.sync_copy(ov.at[pl.ds(0, cp * N_REL)],
                        o_hbm.at[pl.ds(off * N_REL, cp * N_REL)])

    def cloop(ci, _):
        do_chunk(sw + ci * CP, CP)
        return 0
    lax.fori_loop(0, 12, cloop, 0)
    do_chunk(sw + 12 * CP, CP_TAIL)

    @pl.when(wid == 0)
    def _():
        for r in range(N_REL):
            a0 = r * P + PMAIN - _SH[r]
            pltpu.sync_copy(t_hbm.at[pl.ds(a0, PTAIL + _SH[r])],
                            pv[r].at[pl.ds(0, PTAIL + _SH[r])])
        for j in range(2):
            vs = [pv[r][pl.ds(j * 16 + _SH[r], 16)] for r in range(N_REL)]
            for m in range(N_REL):
                mm = j * N_REL + m
                if mm > 6:
                    continue
                pi, pr = _divmod5(ids + 16 * mm)
                pi = jnp.maximum(pi - 16 * j, 0)
                o = jnp.zeros((16,), jnp.float32)
                for r in range(N_REL):
                    g = _vgather(vs[r], pi)
                    o = jnp.where(pr == r, g, o)
                ov[pl.ds(mm * 16, 16)] = o
        pltpu.sync_copy(ov.at[pl.ds(0, PTAIL * N_REL)],
                        o_hbm.at[pl.ds(PMAIN * N_REL, PTAIL * N_REL)])


def _interleave(t_flat):
    return pl.kernel(
        _il_body,
        out_type=jax.ShapeDtypeStruct((P * N_REL,), jnp.float32),
        mesh=_get_mesh(),
        scratch_types=[
            pltpu.VMEM((CP + 16,), jnp.float32),
            pltpu.VMEM((CP + 16,), jnp.float32),
            pltpu.VMEM((CP + 16,), jnp.float32),
            pltpu.VMEM((CP + 16,), jnp.float32),
            pltpu.VMEM((CP + 16,), jnp.float32),
            pltpu.VMEM((CP * N_REL,), jnp.float32),
        ],
    )(t_flat)


def kernel(x, edge_index, edge_type, edge_norm, ord_basis, dense_w,
           basis_matrix, coefs):
    src = jnp.take(x, edge_index[0], axis=0)
    gidx = edge_type * N_NODESP + src
    dst = edge_index[1]
    dst2 = dst + (dst >= USERS).astype(jnp.int32)
    gidx_p = jnp.pad(gidx, (0, EPAD - E))
    dst_p = jnp.pad(dst2, (0, EPAD - E), constant_values=SENT)
    norm_p = jnp.pad(edge_norm, (0, EPAD - E))
    packed = jnp.sort(dst_p * 262144 + jnp.arange(EPAD, dtype=jnp.int32))
    order = packed & 262143
    dst_s = packed >> 18
    gidx_s = jnp.take(gidx_p, order)
    norm_s = jnp.take(norm_p, order)
    b33 = jnp.searchsorted(dst_s, jnp.arange(0, (NW + 1) * RPS, RPS,
                                             dtype=jnp.int32)).astype(jnp.int32)
    bnds = jnp.pad(jnp.stack([b33[:NW], b33[1:]], axis=1),
                   ((0, 1), (0, 6))).reshape(-1)

    ord_pad = jnp.pad(ord_basis.reshape(N_REL, N_NODES, H0),
                      ((0, 0), (0, N_NODESP - N_NODES), (0, H0P - H0)))
    w_pad = jnp.pad(dense_w, ((0, H0P - H0), (0, H1P - H1)))
    basis_pad = jnp.pad(basis_matrix.reshape(2, H1, H1),
                        ((0, 0), (0, H1P - H1), (0, H1P - H1)))

    wtable = _weight_table(ord_pad).reshape(N_REL * N_NODESP, H0P)
    aggr = _rgc(wtable, gidx_s, dst_s, norm_s, bnds)
    t = _decode(coefs, aggr, w_pad, basis_pad)
    out = _interleave(t.reshape(N_REL * P))
    return out.reshape(P, N_REL)


# speedup vs baseline: 1.4683x; 1.0334x over previous
import jax
import jax.numpy as jnp
from jax import lax
from jax.experimental import pallas as pl
from jax.experimental.pallas import tpu as pltpu
from jax.experimental.pallas import tpu_sc as plsc

N_NODES = 2625
N_NODESP = 2632
USERS = 943
ITEMS = N_NODES - USERS
N_REL = 5
H0 = 500
H0P = 512
H1 = 75
H1P = 128
E = 200000

NC, NS = 2, 16
NW = NC * NS
EPAD = 200192
CHUNK = 128

AROWS = 2816
RPS = AROWS // NW
SENT = 4096

P = USERS * ITEMS
PTAIL = 22
PMAIN = P - PTAIL
PW_C = 49568
S31 = PMAIN - PW_C
CP = 4096
CP_TAIL = PW_C - 12 * CP

_GDN = lax.GatherDimensionNumbers(
    offset_dims=(), collapsed_slice_dims=(0,), start_index_map=(0,))


def _vgather(v, idx16):
    return lax.gather(v, idx16[:, None], _GDN, slice_sizes=(1,),
                      mode=lax.GatherScatterMode.PROMISE_IN_BOUNDS)


def _cumsum_body(x_ref, o_ref):
    w = x_ref[0]
    o_ref[0] = w
    for r in range(1, N_REL):
        w = w + x_ref[r]
        o_ref[r] = w


def _weight_table(ord_pad):
    return pl.pallas_call(
        _cumsum_body,
        grid=(7,),
        in_specs=[pl.BlockSpec((N_REL, 376, H0P), lambda i: (0, i, 0))],
        out_specs=pl.BlockSpec((N_REL, 376, H0P), lambda i: (0, i, 0)),
        out_shape=jax.ShapeDtypeStruct((N_REL, N_NODESP, H0P), jnp.float32),
    )(ord_pad)


_sc_mesh = None


def _get_mesh():
    global _sc_mesh
    if _sc_mesh is None:
        _sc_mesh = plsc.VectorSubcoreMesh(
            core_axis_name="c", subcore_axis_name="s",
            num_cores=NC, num_subcores=NS)
    return _sc_mesh


def _rgc_body(w_hbm, gidx_hbm, dst_hbm, norm_hbm, bnds_hbm, aggr_hbm,
              idx_v, dst_v, norm_v, nmk_v, dl_v, rows_v, acc, bv, sem):
    c = lax.axis_index("c")
    s = lax.axis_index("s")
    wid = c * NS + s
    lo = pl.multiple_of(wid * RPS, RPS)
    zero16 = jnp.zeros((16,), jnp.float32)

    def zrow(i, _):
        for k in range(H0P // 16):
            acc[i, pl.ds(k * 16, 16)] = zero16
        return 0
    lax.fori_loop(0, RPS, zrow, 0)

    pltpu.sync_copy(bnds_hbm.at[pl.ds(pl.multiple_of(wid * 8, 8), 16)], bv)
    bb = bv[pl.ds(0, 16)]
    start = bb[0]
    end = bb[1]
    astart = pl.multiple_of((start >> 7) << 7, CHUNK)
    nchunks = (end - astart + 127) >> 7

    def chunk(g, _):
        off = pl.multiple_of(astart + g * CHUNK, CHUNK)
        pltpu.sync_copy(gidx_hbm.at[pl.ds(off, CHUNK)], idx_v)
        pltpu.sync_copy(dst_hbm.at[pl.ds(off, CHUNK)], dst_v)
        pltpu.sync_copy(norm_hbm.at[pl.ds(off, CHUNK)], norm_v)
        pltpu.async_copy(w_hbm.at[idx_v], rows_v, sem).wait()
        for q in range(CHUNK // 16):
            dvec = dst_v[pl.ds(q * 16, 16)]
            nvec = norm_v[pl.ds(q * 16, 16)]
            inm = (dvec >= lo) & (dvec < lo + RPS)
            nmk_v[pl.ds(q * 16, 16)] = jnp.where(inm, nvec, 0.0)
            dl_v[pl.ds(q * 16, 16)] = jnp.clip(dvec - lo, 0, RPS - 1)

        def egroup(g4, _):
            dvec = dl_v[pl.ds(g4 * 16, 16)]
            nvec = nmk_v[pl.ds(g4 * 16, 16)]
            for l in range(16):
                d = dvec[l]
                nv = _vgather(nvec, jnp.full((16,), l, jnp.int32))
                el = g4 * 16 + l

                @plsc.parallel_loop(0, H0P // 16, unroll=8)
                def _(k):
                    off2 = k * 16
                    v = acc[d, pl.ds(off2, 16)]
                    rv = rows_v[el, pl.ds(off2, 16)]
                    acc[d, pl.ds(off2, 16)] = v + rv * nv
            return 0
        lax.fori_loop(0, CHUNK // 16, egroup, 0)
        return 0
    lax.fori_loop(0, nchunks, chunk, 0)

    pltpu.sync_copy(acc, aggr_hbm.at[pl.ds(lo, RPS)])


def _rgc(wtable, gidx_s, dst_s, norm_s, bnds):
    return pl.kernel(
        _rgc_body,
        out_type=jax.ShapeDtypeStruct((AROWS, H0P), jnp.float32),
        mesh=_get_mesh(),
        scratch_types=[
            pltpu.VMEM((CHUNK,), jnp.int32),
            pltpu.VMEM((CHUNK,), jnp.int32),
            pltpu.VMEM((CHUNK,), jnp.float32),
            pltpu.VMEM((CHUNK,), jnp.float32),
            pltpu.VMEM((CHUNK,), jnp.int32),
            pltpu.VMEM((CHUNK, H0P), jnp.float32),
            pltpu.VMEM((RPS, H0P), jnp.float32),
            pltpu.VMEM((16,), jnp.int32),
            pltpu.SemaphoreType.DMA,
        ],
    )(wtable, gidx_s, dst_s, norm_s, bnds)


def _dec_body(coefs_sm, aggr_ref, w_ref, basis_ref, t_ref, feat_sc):
    r = pl.program_id(0)

    @pl.when(r == 0)
    def _():
        f = jnp.maximum(aggr_ref[0:2632, :], 0.0)
        f75 = jnp.dot(f, w_ref[...], preferred_element_type=jnp.float32,
                      precision=lax.Precision.HIGHEST)
        feat_sc[...] = jnp.maximum(f75, 0.0)

    c0 = coefs_sm[r, 0]
    c1 = coefs_sm[r, 1]
    q = c0 * basis_ref[0] + c1 * basis_ref[1]
    u = feat_sc[0:944, :]
    it = feat_sc[944:2632, :]
    zu = jnp.dot(u, q, preferred_element_type=jnp.float32,
                 precision=lax.Precision.HIGHEST)
    t = lax.dot_general(zu, it, (((1,), (1,)), ((), ())),
                        preferred_element_type=jnp.float32,
                        precision=lax.Precision.HIGHEST)
    t_ref[0] = t[0:USERS, 0:ITEMS]


def _decode(coefs, aggr, w_pad, basis_pad):
    return pl.pallas_call(
        _dec_body,
        grid=(N_REL,),
        in_specs=[
            pl.BlockSpec(memory_space=pltpu.SMEM),
            pl.BlockSpec((AROWS, H0P), lambda r: (0, 0)),
            pl.BlockSpec((H0P, H1P), lambda r: (0, 0)),
            pl.BlockSpec((2, H1P, H1P), lambda r: (0, 0, 0)),
        ],
        out_specs=pl.BlockSpec((1, USERS, ITEMS), lambda r: (r, 0, 0)),
        out_shape=jax.ShapeDtypeStruct((N_REL, USERS, ITEMS), jnp.float32),
        scratch_shapes=[pltpu.VMEM((2632, H1P), jnp.float32)],
    )(coefs, aggr, w_pad, basis_pad)


_SH = [(6 * r) % 8 for r in range(N_REL)]


def _divmod5(x16):
    q = (x16 * 52429) >> 18
    return q, x16 - q * N_REL


def _il_body(t_hbm, o_hbm, pv0, pv1, pv2, pv3, pv4, ov):
    pv = [pv0, pv1, pv2, pv3, pv4]
    c = lax.axis_index("c")
    s = lax.axis_index("s")
    wid = c * NS + s
    sw = jnp.minimum(wid * PW_C, S31)
    ids = lax.iota(jnp.int32, 16)
    pi_c, pr_c = [], []
    for m in range(N_REL):
        q, rr = _divmod5(ids + 16 * m)
        pi_c.append(q)
        pr_c.append(rr)

    def do_chunk(off, cp):
        for r in range(N_REL):
            a0 = r * P + off - _SH[r]
            pltpu.sync_copy(t_hbm.at[pl.ds(a0, cp + 16)],
                            pv[r].at[pl.ds(0, cp + 16)])

        def jloop(j, _):
            vs = [pv[r][pl.ds(j * 16 + _SH[r], 16)] for r in range(N_REL)]
            for m in range(N_REL):
                o = jnp.zeros((16,), jnp.float32)
                for r in range(N_REL):
                    g = _vgather(vs[r], pi_c[m])
                    o = jnp.where(pr_c[m] == r, g, o)
                ov[pl.ds(j * 80 + m * 16, 16)] = o
            return 0
        lax.fori_loop(0, cp // 16, jloop, 0)
        pltpu.sync_copy(ov.at[pl.ds(0, cp * N_REL)],
                        o_hbm.at[pl.ds(off * N_REL, cp * N_REL)])

    def cloop(ci, _):
        do_chunk(sw + ci * CP, CP)
        return 0
    lax.fori_loop(0, 12, cloop, 0)
    do_chunk(sw + 12 * CP, CP_TAIL)

    @pl.when(wid == 0)
    def _():
        for r in range(N_REL):
            a0 = r * P + PMAIN - _SH[r]
            pltpu.sync_copy(t_hbm.at[pl.ds(a0, PTAIL + _SH[r])],
                            pv[r].at[pl.ds(0, PTAIL + _SH[r])])
        for j in range(2):
            vs = [pv[r][pl.ds(j * 16 + _SH[r], 16)] for r in range(N_REL)]
            for m in range(N_REL):
                mm = j * N_REL + m
                if mm > 6:
                    continue
                pi, pr = _divmod5(ids + 16 * mm)
                pi = jnp.maximum(pi - 16 * j, 0)
                o = jnp.zeros((16,), jnp.float32)
                for r in range(N_REL):
                    g = _vgather(vs[r], pi)
                    o = jnp.where(pr == r, g, o)
                ov[pl.ds(mm * 16, 16)] = o
        pltpu.sync_copy(ov.at[pl.ds(0, PTAIL * N_REL)],
                        o_hbm.at[pl.ds(PMAIN * N_REL, PTAIL * N_REL)])


def _interleave(t_flat):
    return pl.kernel(
        _il_body,
        out_type=jax.ShapeDtypeStruct((P * N_REL,), jnp.float32),
        mesh=_get_mesh(),
        scratch_types=[
            pltpu.VMEM((CP + 16,), jnp.float32),
            pltpu.VMEM((CP + 16,), jnp.float32),
            pltpu.VMEM((CP + 16,), jnp.float32),
            pltpu.VMEM((CP + 16,), jnp.float32),
            pltpu.VMEM((CP + 16,), jnp.float32),
            pltpu.VMEM((CP * N_REL,), jnp.float32),
        ],
    )(t_flat)


def kernel(x, edge_index, edge_type, edge_norm, ord_basis, dense_w,
           basis_matrix, coefs):
    src = jnp.take(x, edge_index[0], axis=0)
    gidx = edge_type * N_NODESP + src
    dst = edge_index[1]
    dst2 = dst + (dst >= USERS).astype(jnp.int32)
    gidx_p = jnp.pad(gidx, (0, EPAD - E))
    dst_p = jnp.pad(dst2, (0, EPAD - E), constant_values=SENT)
    norm_p = jnp.pad(edge_norm, (0, EPAD - E))
    packed = jnp.sort(dst_p * 262144 + jnp.arange(EPAD, dtype=jnp.int32))
    order = packed & 262143
    dst_s = packed >> 18
    gidx_s = jnp.take(gidx_p, order)
    norm_s = jnp.take(norm_p, order)
    b33 = jnp.searchsorted(dst_s, jnp.arange(0, (NW + 1) * RPS, RPS,
                                             dtype=jnp.int32)).astype(jnp.int32)
    bnds = jnp.pad(jnp.stack([b33[:NW], b33[1:]], axis=1),
                   ((0, 1), (0, 6))).reshape(-1)

    ord_pad = jnp.pad(ord_basis.reshape(N_REL, N_NODES, H0),
                      ((0, 0), (0, N_NODESP - N_NODES), (0, H0P - H0)))
    w_pad = jnp.pad(dense_w, ((0, H0P - H0), (0, H1P - H1)))
    basis_pad = jnp.pad(basis_matrix.reshape(2, H1, H1),
                        ((0, 0), (0, H1P - H1), (0, H1P - H1)))

    wtable = _weight_table(ord_pad).reshape(N_REL * N_NODESP, H0P)
    aggr = _rgc(wtable, gidx_s, dst_s, norm_s, bnds)
    t = _decode(coefs, aggr, w_pad, basis_pad)
    out = _interleave(t.reshape(N_REL * P))
    return out.reshape(P, N_REL)
